# baseline (speedup 1.0000x reference)
import functools

import jax
import jax.numpy as jnp
from jax import lax
from jax.experimental import pallas as pl
from jax.experimental.pallas import tpu as pltpu
from jax.experimental.pallas import tpu_sc as plsc

_C = 96
_G = 256
_B = 2
_NW = 32
_PER_W = (_B * _C) // _NW
_FC = 64
_GR = 72
_EC = 32
_FS = 257
_FPAD = 4624
_FBUF = 15 * 16 * _FS + _FPAD


def _body(cs_hbm, g_hbm, out_hbm, cs_v, gbuf, fbuf, bufe, bufo):
    wid = lax.axis_index("s") * 2 + lax.axis_index("c")
    pltpu.sync_copy(cs_hbm, cs_v)
    lanes = lax.broadcasted_iota(jnp.int32, (16,), 0)
    lanes_fs = lanes * _FS
    cs_vec = cs_v[...]
    s00, s01, s10, s11 = cs_vec[0], cs_vec[1], cs_vec[2], cs_vec[3]

    def pair_body(q, _):
        pair = wid * _PER_W + q
        b = pair // _C
        c = pair - b * _C
        s0 = jnp.where(b == 0, s00, s10)
        s1 = jnp.where(b == 0, s01, s11)

        def f_chunk(jc, _):
            j0 = jc * _FC
            lo = jnp.minimum((jnp.maximum(s1 + j0 - 1, 0) // 8) * 8, _G - _GR)
            pltpu.sync_copy(g_hbm.at[c, pl.ds(lo, _GR), :], gbuf)

            def f_row(jj, _):
                j = j0 + jj
                u = s1 + j
                beta = jnp.minimum(u, 256).astype(jnp.float32) * (1.0 / 256.0)
                r1 = jnp.clip(u - 1, 0, _G - 1) - lo
                r2 = jnp.clip(u, 0, _G - 1) - lo
                bv = jnp.full((16,), beta, jnp.float32)
                bw = 1.0 - bv
                base = j * _FS
                for v in range(16):
                    ga = gbuf[r1, pl.ds(v * 16, 16)]
                    gb = gbuf[r2, pl.ds(v * 16, 16)]
                    fbuf[pl.ds(base + v * 16, 16)] = bv * ga + bw * gb
                return 0

            lax.fori_loop(0, _FC, f_row, 0)
            return 0

        lax.fori_loop(0, 4, f_chunk, 0)
        for v in range(16):
            fbuf[pl.ds(256 * _FS + v * 16, 16)] = gbuf[_GR - 1, pl.ds(v * 16, 16)]

        ob = (b * 4 * _C + c) * (_G * _G)
        def e_chunk(ec, _):
            i0 = ec * _EC

            def e_row(ii, _):
                i = i0 + ii
                t = s0 + i
                alpha = jnp.minimum(t, 256).astype(jnp.float32) * (1.0 / 256.0)
                av = jnp.full((16,), alpha, jnp.float32)
                aw = 1.0 - av
                cav = jnp.full((16,), jnp.clip(t - 1, 0, _G - 1), jnp.int32)
                cbv = jnp.full((16,), jnp.clip(t, 0, _G - 1), jnp.int32)
                ia = lanes_fs + cav
                ib = lanes_fs + cbv
                rowoff = ii * _G
                for v in range(16):
                    basea = v * 16 * _FS
                    xa = plsc.load_gather(fbuf, [ia + basea])
                    xb = plsc.load_gather(fbuf, [ib + basea])
                    bufe[pl.ds(rowoff + v * 16, 16)] = av * xa + aw * xb
                    ya = plsc.load_gather(fbuf, [ia + (basea + _FS)])
                    yb = plsc.load_gather(fbuf, [ib + (basea + _FS)])
                    bufo[pl.ds(rowoff + v * 16, 16)] = av * ya + aw * yb
                return 0

            lax.fori_loop(0, _EC + 1, e_row, 0)
            n = _EC * _G
            pltpu.sync_copy(bufe.at[pl.ds(0, n)],
                            out_hbm.at[pl.ds(ob + i0 * _G, n)])
            pltpu.sync_copy(bufe.at[pl.ds(_G, n)],
                            out_hbm.at[pl.ds(ob + _C * _G * _G + i0 * _G, n)])
            pltpu.sync_copy(bufo.at[pl.ds(0, n)],
                            out_hbm.at[pl.ds(ob + 2 * _C * _G * _G + i0 * _G, n)])
            pltpu.sync_copy(bufo.at[pl.ds(_G, n)],
                            out_hbm.at[pl.ds(ob + 3 * _C * _G * _G + i0 * _G, n)])
            return 0

        lax.fori_loop(0, _G // _EC, e_chunk, 0)
        return 0

    lax.fori_loop(0, _PER_W, pair_body, 0)


@functools.partial(jax.jit, static_argnums=())
def _run(cs_pad, g2):
    mesh = plsc.VectorSubcoreMesh(core_axis_name="c", subcore_axis_name="s",
                                  num_cores=2, num_subcores=16)
    fn = pl.kernel(
        _body,
        out_type=jax.ShapeDtypeStruct((_B * 4 * _C * _G * _G,), jnp.float32),
        mesh=mesh,
        scratch_types=[
            pltpu.VMEM((16,), jnp.int32),
            pltpu.VMEM((_GR, _G), jnp.float32),
            pltpu.VMEM((_FBUF,), jnp.float32),
            pltpu.VMEM(((_EC + 1) * _G,), jnp.float32),
            pltpu.VMEM(((_EC + 1) * _G,), jnp.float32),
        ],
        compiler_params=pltpu.CompilerParams(needs_layout_passes=False),
    )
    return fn(cs_pad, g2).reshape(_B, 4 * _C, _G, _G)


def kernel(coordinate_start, h, w, support_resolution_h, support_resolution_w, grid):
    del h, w, support_resolution_h, support_resolution_w
    cs_pad = jnp.zeros((16,), jnp.int32).at[0:4].set(coordinate_start.reshape(4))
    g2 = grid.reshape(_C, _G, _G)
    return _run(cs_pad, g2)


# speedup vs baseline: 1.3646x; 1.3646x over previous
import functools

import jax
import jax.numpy as jnp
from jax import lax
from jax.experimental import pallas as pl
from jax.experimental.pallas import tpu as pltpu
from jax.experimental.pallas import tpu_sc as plsc

_C = 96
_G = 256
_B = 2
_NW = 32
_PER_W = (_B * _C) // _NW
_FC = 32
_GR = 48
_EC = 16
_FS = 257


def _body(cs_hbm, g_hbm, out_hbm, cs_v, gbuf, fbuf, prev,
          bufa0, bufb0, bufc0, bufd0, bufa1, bufb1, bufc1, bufd1, sem):
    wid = lax.axis_index("s") * 2 + lax.axis_index("c")
    pltpu.sync_copy(cs_hbm, cs_v)
    lanes = lax.broadcasted_iota(jnp.int32, (16,), 0)
    lanes_fs = lanes * _FS
    cs_vec = cs_v[...]
    s00, s01, s10, s11 = cs_vec[0], cs_vec[1], cs_vec[2], cs_vec[3]

    def pair_body(q, _):
        pair = wid * _PER_W + q
        b = pair // _C
        c = pair - b * _C
        s0 = jnp.where(b == 0, s00, s10)
        s1 = jnp.where(b == 0, s01, s11)

        def f_chunk(jc, _):
            j0 = jc * _FC
            lo = jnp.minimum((jnp.maximum(s1 + j0 - 1, 0) // 8) * 8, _G - _GR)
            pltpu.sync_copy(g_hbm.at[c, pl.ds(lo, _GR), :], gbuf)

            def f_row(jj, _):
                j = j0 + jj
                u = s1 + j
                beta = jnp.minimum(u, 256).astype(jnp.float32) * (1.0 / 256.0)
                r1 = jnp.clip(u - 1, 0, _G - 1) - lo
                r2 = jnp.clip(u, 0, _G - 1) - lo
                bv = jnp.full((16,), beta, jnp.float32)
                bw = 1.0 - bv
                base = j * _FS
                for v in range(16):
                    ga = gbuf[r1, pl.ds(v * 16, 16)]
                    gb = gbuf[r2, pl.ds(v * 16, 16)]
                    fbuf[pl.ds(base + v * 16, 16)] = bv * ga + bw * gb
                return 0

            lax.fori_loop(0, _FC, f_row, 0)
            return 0

        lax.fori_loop(0, _G // _FC, f_chunk, 0)
        for v in range(16):
            fbuf[pl.ds(256 * _FS + v * 16, 16)] = gbuf[_GR - 1, pl.ds(v * 16, 16)]

        def e_compute(ec, bufs):
            bufa, bufb, bufc, bufd = bufs
            i0 = ec * _EC

            t0 = s0 + i0
            alpha0 = jnp.minimum(t0, 256).astype(jnp.float32) * (1.0 / 256.0)
            av0 = jnp.full((16,), alpha0, jnp.float32)
            aw0 = 1.0 - av0
            cav0 = jnp.full((16,), jnp.clip(t0 - 1, 0, _G - 1), jnp.int32)
            cbv0 = jnp.full((16,), jnp.clip(t0, 0, _G - 1), jnp.int32)
            for v in range(16):
                basea = lanes_fs + (v * 16 * _FS)
                baseb = basea + _FS
                xa = plsc.load_gather(fbuf, [basea + cav0])
                xb = plsc.load_gather(fbuf, [basea + cbv0])
                prev[pl.ds(v * 16, 16)] = xb
                bufa[0, pl.ds(v * 16, 16)] = av0 * xa + aw0 * xb
                ya = plsc.load_gather(fbuf, [baseb + cav0])
                yb = plsc.load_gather(fbuf, [baseb + cbv0])
                prev[pl.ds(_G + v * 16, 16)] = yb
                bufc[0, pl.ds(v * 16, 16)] = av0 * ya + aw0 * yb

            def e_row(ii, _):
                i = i0 + ii
                t = s0 + i
                alpha = jnp.minimum(t, 256).astype(jnp.float32) * (1.0 / 256.0)
                av = jnp.full((16,), alpha, jnp.float32)
                aw = 1.0 - av
                cbv = jnp.full((16,), jnp.clip(t, 0, _G - 1), jnp.int32)
                for v in range(16):
                    basea = lanes_fs + (v * 16 * _FS)
                    baseb = basea + _FS
                    xa = prev[pl.ds(v * 16, 16)]
                    xb = plsc.load_gather(fbuf, [basea + cbv])
                    prev[pl.ds(v * 16, 16)] = xb
                    ra = av * xa + aw * xb
                    bufa[ii, pl.ds(v * 16, 16)] = ra
                    bufb[ii - 1, pl.ds(v * 16, 16)] = ra
                    ya = prev[pl.ds(_G + v * 16, 16)]
                    yb = plsc.load_gather(fbuf, [baseb + cbv])
                    prev[pl.ds(_G + v * 16, 16)] = yb
                    rb = av * ya + aw * yb
                    bufc[ii, pl.ds(v * 16, 16)] = rb
                    bufd[ii - 1, pl.ds(v * 16, 16)] = rb
                return 0

            lax.fori_loop(1, _EC + 1, e_row, 0)

        def e_copies(ec, bufs):
            i0 = ec * _EC
            for k, buf in enumerate(bufs):
                yield pltpu.make_async_copy(
                    buf.at[pl.ds(0, _EC), :],
                    out_hbm.at[b, k * _C + c, pl.ds(i0, _EC), :],
                    sem,
                )

        def e_start(ec, bufs):
            for cp in e_copies(ec, bufs):
                cp.start()

        def e_drain(ec, bufs):
            for cp in e_copies(ec, bufs):
                cp.wait()

        set0 = (bufa0, bufb0, bufc0, bufd0)
        set1 = (bufa1, bufb1, bufc1, bufd1)

        e_compute(0, set0)
        e_start(0, set0)
        e_compute(1, set1)
        e_start(1, set1)

        def e_ring(cp, _):
            ec = 2 * cp
            e_drain(ec - 2, set0)
            e_compute(ec, set0)
            e_start(ec, set0)
            e_drain(ec - 1, set1)
            e_compute(ec + 1, set1)
            e_start(ec + 1, set1)
            return 0

        lax.fori_loop(1, (_G // _EC) // 2, e_ring, 0)
        e_drain(_G // _EC - 2, set0)
        e_drain(_G // _EC - 1, set1)
        return 0

    lax.fori_loop(0, _PER_W, pair_body, 0)


@functools.partial(jax.jit, static_argnums=())
def _run(cs_pad, g2):
    mesh = plsc.VectorSubcoreMesh(core_axis_name="c", subcore_axis_name="s",
                                  num_cores=2, num_subcores=16)
    fn = pl.kernel(
        _body,
        out_type=jax.ShapeDtypeStruct((_B, 4 * _C, _G, _G), jnp.float32),
        mesh=mesh,
        scratch_types=[
            pltpu.VMEM((16,), jnp.int32),
            pltpu.VMEM((_GR, _G), jnp.float32),
            pltpu.VMEM((257 * _FS,), jnp.float32),
            pltpu.VMEM((2 * _G,), jnp.float32),
            pltpu.VMEM((_EC + 1, _G), jnp.float32),
            pltpu.VMEM((_EC + 1, _G), jnp.float32),
            pltpu.VMEM((_EC + 1, _G), jnp.float32),
            pltpu.VMEM((_EC + 1, _G), jnp.float32),
            pltpu.VMEM((_EC + 1, _G), jnp.float32),
            pltpu.VMEM((_EC + 1, _G), jnp.float32),
            pltpu.VMEM((_EC + 1, _G), jnp.float32),
            pltpu.VMEM((_EC + 1, _G), jnp.float32),
            pltpu.SemaphoreType.DMA,
        ],
        compiler_params=pltpu.CompilerParams(needs_layout_passes=False),
    )
    return fn(cs_pad, g2)


def kernel(coordinate_start, h, w, support_resolution_h, support_resolution_w, grid):
    del h, w, support_resolution_h, support_resolution_w
    cs_pad = jnp.zeros((16,), jnp.int32).at[0:4].set(coordinate_start.reshape(4))
    g2 = grid.reshape(_C, _G, _G)
    return _run(cs_pad, g2)


# speedup vs baseline: 1.7773x; 1.3024x over previous
import functools

import jax
import jax.numpy as jnp
from jax import lax
from jax.experimental import pallas as pl
from jax.experimental.pallas import tpu as pltpu
from jax.experimental.pallas import tpu_sc as plsc

_C = 96
_G = 256
_B = 2
_NW = 32
_PER_W = (_B * _C) // _NW
_FC = 32
_GR = 48
_EC = 16
_FS = 257
_FPAD = 4368


def _body(cs_hbm, g_hbm, out_hbm, cs_v, gbuf, fbuf,
          bufa0, bufb0, bufc0, bufd0, bufa1, bufb1, bufc1, bufd1, sem):
    wid = lax.axis_index("s") * 2 + lax.axis_index("c")
    pltpu.sync_copy(cs_hbm, cs_v)
    lanes = lax.broadcasted_iota(jnp.int32, (16,), 0)
    lanes_fs = lanes * _FS
    cs_vec = cs_v[...]
    s00, s01, s10, s11 = cs_vec[0], cs_vec[1], cs_vec[2], cs_vec[3]

    def pair_body(q, _):
        pair = wid * _PER_W + q
        b = pair // _C
        c = pair - b * _C
        s0 = jnp.where(b == 0, s00, s10)
        s1 = jnp.where(b == 0, s01, s11)

        def f_chunk(jc, _):
            j0 = jc * _FC
            lo = jnp.minimum((jnp.maximum(s1 + j0 - 1, 0) // 8) * 8, _G - _GR)
            pltpu.sync_copy(g_hbm.at[c, pl.ds(lo, _GR), :], gbuf)

            def f_row(jj, _):
                j = j0 + jj
                u = s1 + j
                beta = jnp.minimum(u, 256).astype(jnp.float32) * (1.0 / 256.0)
                r1 = jnp.clip(u - 1, 0, _G - 1) - lo
                r2 = jnp.clip(u, 0, _G - 1) - lo
                bv = jnp.full((16,), beta, jnp.float32)
                bw = 1.0 - bv
                base = j * _FS
                for v in range(16):
                    ga = gbuf[r1, pl.ds(v * 16, 16)]
                    gb = gbuf[r2, pl.ds(v * 16, 16)]
                    fbuf[pl.ds(base + v * 16, 16)] = bv * ga + bw * gb
                return 0

            lax.fori_loop(0, _FC, f_row, 0)
            return 0

        lax.fori_loop(0, _G // _FC, f_chunk, 0)
        for v in range(16):
            fbuf[pl.ds(256 * _FS + v * 16, 16)] = gbuf[_GR - 1, pl.ds(v * 16, 16)]

        def e_compute(ec, bufs):
            bufa, bufb, bufc, bufd = bufs
            i0 = ec * _EC

            t0 = s0 + i0
            alpha0 = jnp.minimum(t0, 256).astype(jnp.float32) * (1.0 / 256.0)
            av0 = jnp.full((16,), alpha0, jnp.float32)
            aw0 = 1.0 - av0
            ia0 = lanes_fs + jnp.full((16,), jnp.clip(t0 - 1, 0, _G - 1), jnp.int32)
            ib0 = lanes_fs + jnp.full((16,), jnp.clip(t0, 0, _G - 1), jnp.int32)
            iay0 = ia0 + _FS
            iby0 = ib0 + _FS
            for v in range(16):
                fa = fbuf.at[pl.ds(v * 16 * _FS, _FPAD)]
                xa = plsc.load_gather(fa, [ia0])
                xb = plsc.load_gather(fa, [ib0])
                bufa[0, pl.ds(v * 16, 16)] = av0 * xa + aw0 * xb
                ya = plsc.load_gather(fa, [iay0])
                yb = plsc.load_gather(fa, [iby0])
                bufc[0, pl.ds(v * 16, 16)] = av0 * ya + aw0 * yb

            def e_rowpair(p, _):
                ii = 1 + 2 * p
                t = s0 + i0 + ii
                a1 = jnp.minimum(t, 256).astype(jnp.float32) * (1.0 / 256.0)
                a2 = jnp.minimum(t + 1, 256).astype(jnp.float32) * (1.0 / 256.0)
                av1 = jnp.full((16,), a1, jnp.float32)
                aw1 = 1.0 - av1
                av2 = jnp.full((16,), a2, jnp.float32)
                aw2 = 1.0 - av2
                j0v = lanes_fs + jnp.full((16,), jnp.clip(t - 1, 0, _G - 1), jnp.int32)
                j1v = lanes_fs + jnp.full((16,), jnp.clip(t, 0, _G - 1), jnp.int32)
                j2v = lanes_fs + jnp.full((16,), jnp.clip(t + 1, 0, _G - 1), jnp.int32)
                k0v = j0v + _FS
                k1v = j1v + _FS
                k2v = j2v + _FS
                for v in range(16):
                    fa = fbuf.at[pl.ds(v * 16 * _FS, _FPAD)]
                    x0 = plsc.load_gather(fa, [j0v])
                    x1 = plsc.load_gather(fa, [j1v])
                    x2 = plsc.load_gather(fa, [j2v])
                    r1 = av1 * x0 + aw1 * x1
                    r2 = av2 * x1 + aw2 * x2
                    bufa[ii, pl.ds(v * 16, 16)] = r1
                    bufb[ii - 1, pl.ds(v * 16, 16)] = r1
                    bufa[ii + 1, pl.ds(v * 16, 16)] = r2
                    bufb[ii, pl.ds(v * 16, 16)] = r2
                    y0 = plsc.load_gather(fa, [k0v])
                    y1 = plsc.load_gather(fa, [k1v])
                    y2 = plsc.load_gather(fa, [k2v])
                    u1 = av1 * y0 + aw1 * y1
                    u2 = av2 * y1 + aw2 * y2
                    bufc[ii, pl.ds(v * 16, 16)] = u1
                    bufd[ii - 1, pl.ds(v * 16, 16)] = u1
                    bufc[ii + 1, pl.ds(v * 16, 16)] = u2
                    bufd[ii, pl.ds(v * 16, 16)] = u2
                return 0

            lax.fori_loop(0, _EC // 2, e_rowpair, 0)

        def e_copies(ec, bufs):
            i0 = ec * _EC
            for k, buf in enumerate(bufs):
                yield pltpu.make_async_copy(
                    buf.at[pl.ds(0, _EC), :],
                    out_hbm.at[b, k * _C + c, pl.ds(i0, _EC), :],
                    sem,
                )

        def e_start(ec, bufs):
            for cp in e_copies(ec, bufs):
                cp.start()

        def e_drain(ec, bufs):
            for cp in e_copies(ec, bufs):
                cp.wait()

        set0 = (bufa0, bufb0, bufc0, bufd0)
        set1 = (bufa1, bufb1, bufc1, bufd1)

        e_compute(0, set0)
        e_start(0, set0)
        e_compute(1, set1)
        e_start(1, set1)

        def e_ring(cp, _):
            ec = 2 * cp
            e_drain(ec - 2, set0)
            e_compute(ec, set0)
            e_start(ec, set0)
            e_drain(ec - 1, set1)
            e_compute(ec + 1, set1)
            e_start(ec + 1, set1)
            return 0

        lax.fori_loop(1, (_G // _EC) // 2, e_ring, 0)
        e_drain(_G // _EC - 2, set0)
        e_drain(_G // _EC - 1, set1)
        return 0

    lax.fori_loop(0, _PER_W, pair_body, 0)


@functools.partial(jax.jit, static_argnums=())
def _run(cs_pad, g2):
    mesh = plsc.VectorSubcoreMesh(core_axis_name="c", subcore_axis_name="s",
                                  num_cores=2, num_subcores=16)
    fn = pl.kernel(
        _body,
        out_type=jax.ShapeDtypeStruct((_B, 4 * _C, _G, _G), jnp.float32),
        mesh=mesh,
        scratch_types=[
            pltpu.VMEM((16,), jnp.int32),
            pltpu.VMEM((_GR, _G), jnp.float32),
            pltpu.VMEM((257 * _FS,), jnp.float32),
            pltpu.VMEM((_EC + 1, _G), jnp.float32),
            pltpu.VMEM((_EC + 1, _G), jnp.float32),
            pltpu.VMEM((_EC + 1, _G), jnp.float32),
            pltpu.VMEM((_EC + 1, _G), jnp.float32),
            pltpu.VMEM((_EC + 1, _G), jnp.float32),
            pltpu.VMEM((_EC + 1, _G), jnp.float32),
            pltpu.VMEM((_EC + 1, _G), jnp.float32),
            pltpu.VMEM((_EC + 1, _G), jnp.float32),
            pltpu.SemaphoreType.DMA,
        ],
        compiler_params=pltpu.CompilerParams(needs_layout_passes=False),
    )
    return fn(cs_pad, g2)


def kernel(coordinate_start, h, w, support_resolution_h, support_resolution_w, grid):
    del h, w, support_resolution_h, support_resolution_w
    cs_pad = jnp.zeros((16,), jnp.int32).at[0:4].set(coordinate_start.reshape(4))
    g2 = grid.reshape(_C, _G, _G)
    return _run(cs_pad, g2)


# speedup vs baseline: 1.8719x; 1.0532x over previous
import functools

import jax
import jax.numpy as jnp
from jax import lax
from jax.experimental import pallas as pl
from jax.experimental.pallas import tpu as pltpu
from jax.experimental.pallas import tpu_sc as plsc

_C = 96
_G = 256
_B = 2
_NW = 32
_PER_W = (_B * _C) // _NW
_FC = 32
_GR = 48
_EC = 16
_FS = 257
_FPAD = 4368


def _body(cs_hbm, g_hbm, out_hbm, cs_v, gbuf0, gbuf1, fbuf,
          bufa0, bufb0, bufc0, bufd0, bufa1, bufb1, bufc1, bufd1, sem):
    wid = lax.axis_index("s") * 2 + lax.axis_index("c")
    pltpu.sync_copy(cs_hbm, cs_v)
    lanes = lax.broadcasted_iota(jnp.int32, (16,), 0)
    lanes_fs = lanes * _FS
    cs_vec = cs_v[...]
    s00, s01, s10, s11 = cs_vec[0], cs_vec[1], cs_vec[2], cs_vec[3]

    def pair_body(q, _):
        pair = wid * _PER_W + q
        b = pair // _C
        c = pair - b * _C
        s0 = jnp.where(b == 0, s00, s10)
        s1 = jnp.where(b == 0, s01, s11)

        def g_lo(jc):
            j0 = jc * _FC
            return jnp.minimum((jnp.maximum(s1 + j0 - 1, 0) // 8) * 8,
                               _G - _GR)

        def g_copy(jc, gb):
            return pltpu.make_async_copy(
                g_hbm.at[c, pl.ds(g_lo(jc), _GR), :], gb, sem)

        def f_chunk(jc, gb):
            j0 = jc * _FC
            lo = g_lo(jc)

            def f_row(jj, _):
                j = j0 + jj
                u = s1 + j
                beta = jnp.minimum(u, 256).astype(jnp.float32) * (1.0 / 256.0)
                r1 = jnp.clip(u - 1, 0, _G - 1) - lo
                r2 = jnp.clip(u, 0, _G - 1) - lo
                bv = jnp.full((16,), beta, jnp.float32)
                bw = 1.0 - bv
                base = j * _FS
                for v in range(16):
                    ga = gb[r1, pl.ds(v * 16, 16)]
                    gb_ = gb[r2, pl.ds(v * 16, 16)]
                    fbuf[pl.ds(base + v * 16, 16)] = bv * ga + bw * gb_
                return 0

            lax.fori_loop(0, _FC, f_row, 0)

        g_copy(0, gbuf0).start()
        for jc in range(_G // _FC):
            gcur = gbuf0 if jc % 2 == 0 else gbuf1
            gnxt = gbuf1 if jc % 2 == 0 else gbuf0
            g_copy(jc, gcur).wait()
            if jc + 1 < _G // _FC:
                g_copy(jc + 1, gnxt).start()
            f_chunk(jc, gcur)
        gl = gbuf0 if (_G // _FC - 1) % 2 == 0 else gbuf1
        for v in range(16):
            fbuf[pl.ds(256 * _FS + v * 16, 16)] = gl[_GR - 1, pl.ds(v * 16, 16)]

        def e_compute(ec, bufs):
            bufa, bufb, bufc, bufd = bufs
            i0 = ec * _EC

            t0 = s0 + i0
            alpha0 = jnp.minimum(t0, 256).astype(jnp.float32) * (1.0 / 256.0)
            av0 = jnp.full((16,), alpha0, jnp.float32)
            aw0 = 1.0 - av0
            ia0 = lanes_fs + jnp.full((16,), jnp.clip(t0 - 1, 0, _G - 1), jnp.int32)
            ib0 = lanes_fs + jnp.full((16,), jnp.clip(t0, 0, _G - 1), jnp.int32)
            iay0 = ia0 + _FS
            iby0 = ib0 + _FS
            for v in range(16):
                fa = fbuf.at[pl.ds(v * 16 * _FS, _FPAD)]
                xa = plsc.load_gather(fa, [ia0])
                xb = plsc.load_gather(fa, [ib0])
                bufa[0, pl.ds(v * 16, 16)] = av0 * xa + aw0 * xb
                ya = plsc.load_gather(fa, [iay0])
                yb = plsc.load_gather(fa, [iby0])
                bufc[0, pl.ds(v * 16, 16)] = av0 * ya + aw0 * yb

            def pair_taps(p):
                ii = 1 + 2 * p
                t = s0 + i0 + ii
                a1 = jnp.minimum(t, 256).astype(jnp.float32) * (1.0 / 256.0)
                a2 = jnp.minimum(t + 1, 256).astype(jnp.float32) * (1.0 / 256.0)
                av1 = jnp.full((16,), a1, jnp.float32)
                aw1 = 1.0 - av1
                av2 = jnp.full((16,), a2, jnp.float32)
                aw2 = 1.0 - av2
                j0v = lanes_fs + jnp.full((16,), jnp.clip(t - 1, 0, _G - 1), jnp.int32)
                j1v = lanes_fs + jnp.full((16,), jnp.clip(t, 0, _G - 1), jnp.int32)
                j2v = lanes_fs + jnp.full((16,), jnp.clip(t + 1, 0, _G - 1), jnp.int32)
                k0v = j0v + _FS
                k1v = j1v + _FS
                k2v = j2v + _FS
                return ii, av1, aw1, av2, aw2, j0v, j1v, j2v, k0v, k1v, k2v

            def e_rowpair(p, _):
                ii, av1, aw1, av2, aw2, j0v, j1v, j2v, k0v, k1v, k2v = pair_taps(p)
                for v in range(16):
                    fa = fbuf.at[pl.ds(v * 16 * _FS, _FPAD)]
                    x0 = plsc.load_gather(fa, [j0v])
                    x1 = plsc.load_gather(fa, [j1v])
                    x2 = plsc.load_gather(fa, [j2v])
                    r1 = av1 * x0 + aw1 * x1
                    r2 = av2 * x1 + aw2 * x2
                    bufa[ii, pl.ds(v * 16, 16)] = r1
                    bufb[ii - 1, pl.ds(v * 16, 16)] = r1
                    bufa[ii + 1, pl.ds(v * 16, 16)] = r2
                    bufb[ii, pl.ds(v * 16, 16)] = r2
                    y0 = plsc.load_gather(fa, [k0v])
                    y1 = plsc.load_gather(fa, [k1v])
                    y2 = plsc.load_gather(fa, [k2v])
                    u1 = av1 * y0 + aw1 * y1
                    u2 = av2 * y1 + aw2 * y2
                    bufc[ii, pl.ds(v * 16, 16)] = u1
                    bufd[ii - 1, pl.ds(v * 16, 16)] = u1
                    bufc[ii + 1, pl.ds(v * 16, 16)] = u2
                    bufd[ii, pl.ds(v * 16, 16)] = u2
                return 0

            lax.fori_loop(0, _EC // 2 - 1, e_rowpair, 0)

            ii, av1, aw1, av2, aw2, j0v, j1v, j2v, k0v, k1v, k2v = pair_taps(
                _EC // 2 - 1)
            for v in range(16):
                fa = fbuf.at[pl.ds(v * 16 * _FS, _FPAD)]
                x0 = plsc.load_gather(fa, [j0v])
                x1 = plsc.load_gather(fa, [j1v])
                x2 = plsc.load_gather(fa, [j2v])
                r1 = av1 * x0 + aw1 * x1
                r2 = av2 * x1 + aw2 * x2
                bufa[_EC - 1, pl.ds(v * 16, 16)] = r1
                bufb[_EC - 2, pl.ds(v * 16, 16)] = r1
                bufb[_EC - 1, pl.ds(v * 16, 16)] = r2
                y0 = plsc.load_gather(fa, [k0v])
                y1 = plsc.load_gather(fa, [k1v])
                y2 = plsc.load_gather(fa, [k2v])
                u1 = av1 * y0 + aw1 * y1
                u2 = av2 * y1 + aw2 * y2
                bufc[_EC - 1, pl.ds(v * 16, 16)] = u1
                bufd[_EC - 2, pl.ds(v * 16, 16)] = u1
                bufd[_EC - 1, pl.ds(v * 16, 16)] = u2

        def e_copies(ec, bufs):
            i0 = ec * _EC
            for k, buf in enumerate(bufs):
                yield pltpu.make_async_copy(
                    buf.at[pl.ds(0, _EC), :],
                    out_hbm.at[b, k * _C + c, pl.ds(i0, _EC), :],
                    sem,
                )

        def e_start(ec, bufs):
            for cp in e_copies(ec, bufs):
                cp.start()

        def e_drain(ec, bufs):
            for cp in e_copies(ec, bufs):
                cp.wait()

        set0 = (bufa0, bufb0, bufc0, bufd0)
        set1 = (bufa1, bufb1, bufc1, bufd1)

        e_compute(0, set0)
        e_start(0, set0)
        e_compute(1, set1)
        e_start(1, set1)

        def e_ring(cp, _):
            ec = 2 * cp
            e_drain(ec - 2, set0)
            e_compute(ec, set0)
            e_start(ec, set0)
            e_drain(ec - 1, set1)
            e_compute(ec + 1, set1)
            e_start(ec + 1, set1)
            return 0

        lax.fori_loop(1, (_G // _EC) // 2, e_ring, 0)
        e_drain(_G // _EC - 2, set0)
        e_drain(_G // _EC - 1, set1)
        return 0

    lax.fori_loop(0, _PER_W, pair_body, 0)


@functools.partial(jax.jit, static_argnums=())
def _run(cs_pad, g2):
    mesh = plsc.VectorSubcoreMesh(core_axis_name="c", subcore_axis_name="s",
                                  num_cores=2, num_subcores=16)
    fn = pl.kernel(
        _body,
        out_type=jax.ShapeDtypeStruct((_B, 4 * _C, _G, _G), jnp.float32),
        mesh=mesh,
        scratch_types=[
            pltpu.VMEM((16,), jnp.int32),
            pltpu.VMEM((_GR, _G), jnp.float32),
            pltpu.VMEM((_GR, _G), jnp.float32),
            pltpu.VMEM((257 * _FS,), jnp.float32),
            pltpu.VMEM((_EC, _G), jnp.float32),
            pltpu.VMEM((_EC, _G), jnp.float32),
            pltpu.VMEM((_EC, _G), jnp.float32),
            pltpu.VMEM((_EC, _G), jnp.float32),
            pltpu.VMEM((_EC, _G), jnp.float32),
            pltpu.VMEM((_EC, _G), jnp.float32),
            pltpu.VMEM((_EC, _G), jnp.float32),
            pltpu.VMEM((_EC, _G), jnp.float32),
            pltpu.SemaphoreType.DMA,
        ],
        compiler_params=pltpu.CompilerParams(needs_layout_passes=False),
    )
    return fn(cs_pad, g2)


def kernel(coordinate_start, h, w, support_resolution_h, support_resolution_w, grid):
    del h, w, support_resolution_h, support_resolution_w
    cs_pad = jnp.zeros((16,), jnp.int32).at[0:4].set(coordinate_start.reshape(4))
    g2 = grid.reshape(_C, _G, _G)
    return _run(cs_pad, g2)


# speedup vs baseline: 2.2190x; 1.1854x over previous
import functools

import jax
import jax.numpy as jnp
from jax import lax
from jax.experimental import pallas as pl
from jax.experimental.pallas import tpu as pltpu
from jax.experimental.pallas import tpu_sc as plsc

_C = 96
_G = 256
_B = 2
_NW = 32
_PER_W = (_B * _C) // _NW
_FC = 32
_GR = 48
_EC = 16
_FS = 257
_FPAD = 4368


def _body(cs_hbm, g_hbm, out_hbm, cs_v, gbuf0, gbuf1, fbuf,
          bufa0, bufb0, bufc0, bufd0, bufa1, bufb1, bufc1, bufd1, sem):
    wid = lax.axis_index("s") * 2 + lax.axis_index("c")
    pltpu.sync_copy(cs_hbm, cs_v)
    lanes = lax.broadcasted_iota(jnp.int32, (16,), 0)
    lanes_fs = lanes * _FS
    cs_vec = cs_v[...]
    s00, s01, s10, s11 = cs_vec[0], cs_vec[1], cs_vec[2], cs_vec[3]

    def pair_body(q, _):
        pair = wid * _PER_W + q
        b = pair // _C
        c = pair - b * _C
        s0 = jnp.where(b == 0, s00, s10)
        s1 = jnp.where(b == 0, s01, s11)

        def g_lo(jc):
            j0 = jc * _FC
            return jnp.minimum((jnp.maximum(s1 + j0 - 1, 0) // 8) * 8,
                               _G - _GR)

        def g_copy(jc, gb):
            return pltpu.make_async_copy(
                g_hbm.at[c, pl.ds(g_lo(jc), _GR), :], gb, sem)

        def f_chunk(jc, gb):
            j0 = jc * _FC
            lo = g_lo(jc)

            def f_rowpair(jp, _):
                j = j0 + 2 * jp
                u = s1 + j
                b1 = jnp.minimum(u, 256).astype(jnp.float32) * (1.0 / 256.0)
                b2 = jnp.minimum(u + 1, 256).astype(jnp.float32) * (1.0 / 256.0)
                bv1 = jnp.full((16,), b1, jnp.float32)
                bw1 = 1.0 - bv1
                bv2 = jnp.full((16,), b2, jnp.float32)
                bw2 = 1.0 - bv2
                r0 = jnp.clip(u - 1, 0, _G - 1) - lo
                r1 = jnp.clip(u, 0, _G - 1) - lo
                r2 = jnp.clip(u + 1, 0, _G - 1) - lo
                base = j * _FS
                for v in range(16):
                    g0 = gb[r0, pl.ds(v * 16, 16)]
                    g1 = gb[r1, pl.ds(v * 16, 16)]
                    g2 = gb[r2, pl.ds(v * 16, 16)]
                    fbuf[pl.ds(base + v * 16, 16)] = bv1 * g0 + bw1 * g1
                    fbuf[pl.ds(base + _FS + v * 16, 16)] = bv2 * g1 + bw2 * g2
                return 0

            lax.fori_loop(0, _FC // 2, f_rowpair, 0)

        g_copy(0, gbuf0).start()
        for jc in range(_G // _FC):
            gcur = gbuf0 if jc % 2 == 0 else gbuf1
            gnxt = gbuf1 if jc % 2 == 0 else gbuf0
            g_copy(jc, gcur).wait()
            if jc + 1 < _G // _FC:
                g_copy(jc + 1, gnxt).start()
            f_chunk(jc, gcur)
        gl = gbuf0 if (_G // _FC - 1) % 2 == 0 else gbuf1
        for v in range(16):
            fbuf[pl.ds(256 * _FS + v * 16, 16)] = gl[_GR - 1, pl.ds(v * 16, 16)]

        def e_compute(ec, bufs):
            bufa, bufb, bufc, bufd = bufs
            i0 = ec * _EC

            t0 = s0 + i0
            alpha0 = jnp.minimum(t0, 256).astype(jnp.float32) * (1.0 / 256.0)
            av0 = jnp.full((16,), alpha0, jnp.float32)
            aw0 = 1.0 - av0
            ia0 = lanes_fs + jnp.full((16,), jnp.clip(t0 - 1, 0, _G - 1), jnp.int32)
            ib0 = lanes_fs + jnp.full((16,), jnp.clip(t0, 0, _G - 1), jnp.int32)
            iay0 = ia0 + _FS
            iby0 = ib0 + _FS
            for v in range(16):
                fa = fbuf.at[pl.ds(v * 16 * _FS, _FPAD)]
                xa = plsc.load_gather(fa, [ia0])
                xb = plsc.load_gather(fa, [ib0])
                bufa[0, pl.ds(v * 16, 16)] = av0 * xa + aw0 * xb
                ya = plsc.load_gather(fa, [iay0])
                yb = plsc.load_gather(fa, [iby0])
                bufc[0, pl.ds(v * 16, 16)] = av0 * ya + aw0 * yb

            def pair_taps(p):
                ii = 1 + 2 * p
                t = s0 + i0 + ii
                a1 = jnp.minimum(t, 256).astype(jnp.float32) * (1.0 / 256.0)
                a2 = jnp.minimum(t + 1, 256).astype(jnp.float32) * (1.0 / 256.0)
                av1 = jnp.full((16,), a1, jnp.float32)
                aw1 = 1.0 - av1
                av2 = jnp.full((16,), a2, jnp.float32)
                aw2 = 1.0 - av2
                j0v = lanes_fs + jnp.full((16,), jnp.clip(t - 1, 0, _G - 1), jnp.int32)
                j1v = lanes_fs + jnp.full((16,), jnp.clip(t, 0, _G - 1), jnp.int32)
                j2v = lanes_fs + jnp.full((16,), jnp.clip(t + 1, 0, _G - 1), jnp.int32)
                k0v = j0v + _FS
                k1v = j1v + _FS
                k2v = j2v + _FS
                return ii, av1, aw1, av2, aw2, j0v, j1v, j2v, k0v, k1v, k2v

            def e_rowpair(p, _):
                ii, av1, aw1, av2, aw2, j0v, j1v, j2v, k0v, k1v, k2v = pair_taps(p)
                for v in range(16):
                    fa = fbuf.at[pl.ds(v * 16 * _FS, _FPAD)]
                    x0 = plsc.load_gather(fa, [j0v])
                    x1 = plsc.load_gather(fa, [j1v])
                    x2 = plsc.load_gather(fa, [j2v])
                    r1 = av1 * x0 + aw1 * x1
                    r2 = av2 * x1 + aw2 * x2
                    bufa[ii, pl.ds(v * 16, 16)] = r1
                    bufb[ii - 1, pl.ds(v * 16, 16)] = r1
                    bufa[ii + 1, pl.ds(v * 16, 16)] = r2
                    bufb[ii, pl.ds(v * 16, 16)] = r2
                    y0 = plsc.load_gather(fa, [k0v])
                    y1 = plsc.load_gather(fa, [k1v])
                    y2 = plsc.load_gather(fa, [k2v])
                    u1 = av1 * y0 + aw1 * y1
                    u2 = av2 * y1 + aw2 * y2
                    bufc[ii, pl.ds(v * 16, 16)] = u1
                    bufd[ii - 1, pl.ds(v * 16, 16)] = u1
                    bufc[ii + 1, pl.ds(v * 16, 16)] = u2
                    bufd[ii, pl.ds(v * 16, 16)] = u2
                return 0

            lax.fori_loop(0, _EC // 2 - 1, e_rowpair, 0)

            ii, av1, aw1, av2, aw2, j0v, j1v, j2v, k0v, k1v, k2v = pair_taps(
                _EC // 2 - 1)
            for v in range(16):
                fa = fbuf.at[pl.ds(v * 16 * _FS, _FPAD)]
                x0 = plsc.load_gather(fa, [j0v])
                x1 = plsc.load_gather(fa, [j1v])
                x2 = plsc.load_gather(fa, [j2v])
                r1 = av1 * x0 + aw1 * x1
                r2 = av2 * x1 + aw2 * x2
                bufa[_EC - 1, pl.ds(v * 16, 16)] = r1
                bufb[_EC - 2, pl.ds(v * 16, 16)] = r1
                bufb[_EC - 1, pl.ds(v * 16, 16)] = r2
                y0 = plsc.load_gather(fa, [k0v])
                y1 = plsc.load_gather(fa, [k1v])
                y2 = plsc.load_gather(fa, [k2v])
                u1 = av1 * y0 + aw1 * y1
                u2 = av2 * y1 + aw2 * y2
                bufc[_EC - 1, pl.ds(v * 16, 16)] = u1
                bufd[_EC - 2, pl.ds(v * 16, 16)] = u1
                bufd[_EC - 1, pl.ds(v * 16, 16)] = u2

        def e_copies(ec, bufs):
            i0 = ec * _EC
            for k, buf in enumerate(bufs):
                yield pltpu.make_async_copy(
                    buf.at[pl.ds(0, _EC), :],
                    out_hbm.at[b, k * _C + c, pl.ds(i0, _EC), :],
                    sem,
                )

        def e_start(ec, bufs):
            for cp in e_copies(ec, bufs):
                cp.start()

        def e_drain(ec, bufs):
            for cp in e_copies(ec, bufs):
                cp.wait()

        set0 = (bufa0, bufb0, bufc0, bufd0)
        set1 = (bufa1, bufb1, bufc1, bufd1)

        e_compute(0, set0)
        e_start(0, set0)
        e_compute(1, set1)
        e_start(1, set1)

        def e_ring(cp, _):
            ec = 2 * cp
            e_drain(ec - 2, set0)
            e_compute(ec, set0)
            e_start(ec, set0)
            e_drain(ec - 1, set1)
            e_compute(ec + 1, set1)
            e_start(ec + 1, set1)
            return 0

        lax.fori_loop(1, (_G // _EC) // 2, e_ring, 0)
        e_drain(_G // _EC - 2, set0)
        e_drain(_G // _EC - 1, set1)
        return 0

    lax.fori_loop(0, _PER_W, pair_body, 0)


@functools.partial(jax.jit, static_argnums=())
def _run(cs_pad, g2):
    mesh = plsc.VectorSubcoreMesh(core_axis_name="c", subcore_axis_name="s",
                                  num_cores=2, num_subcores=16)
    fn = pl.kernel(
        _body,
        out_type=jax.ShapeDtypeStruct((_B, 4 * _C, _G, _G), jnp.float32),
        mesh=mesh,
        scratch_types=[
            pltpu.VMEM((16,), jnp.int32),
            pltpu.VMEM((_GR, _G), jnp.float32),
            pltpu.VMEM((_GR, _G), jnp.float32),
            pltpu.VMEM((257 * _FS,), jnp.float32),
            pltpu.VMEM((_EC, _G), jnp.float32),
            pltpu.VMEM((_EC, _G), jnp.float32),
            pltpu.VMEM((_EC, _G), jnp.float32),
            pltpu.VMEM((_EC, _G), jnp.float32),
            pltpu.VMEM((_EC, _G), jnp.float32),
            pltpu.VMEM((_EC, _G), jnp.float32),
            pltpu.VMEM((_EC, _G), jnp.float32),
            pltpu.VMEM((_EC, _G), jnp.float32),
            pltpu.SemaphoreType.DMA,
        ],
        compiler_params=pltpu.CompilerParams(needs_layout_passes=False),
    )
    return fn(cs_pad, g2)


def kernel(coordinate_start, h, w, support_resolution_h, support_resolution_w, grid):
    del h, w, support_resolution_h, support_resolution_w
    cs_pad = jnp.zeros((16,), jnp.int32).at[0:4].set(coordinate_start.reshape(4))
    g2 = grid.reshape(_C, _G, _G)
    return _run(cs_pad, g2)


# speedup vs baseline: 2.2582x; 1.0177x over previous
import functools

import jax
import jax.numpy as jnp
from jax import lax
from jax.experimental import pallas as pl
from jax.experimental.pallas import tpu as pltpu
from jax.experimental.pallas import tpu_sc as plsc

_C = 96
_G = 256
_B = 2
_NW = 32
_PER_W = (_B * _C) // _NW
_FC = 32
_GR = 48
_EC = 16
_FS = 257
_FPAD = 4368


def _body(cs_hbm, g_hbm, out_hbm, cs_v, gbuf0, gbuf1, fbuf,
          bufa0, bufb0, bufc0, bufd0, bufa1, bufb1, bufc1, bufd1, sem):
    wid = lax.axis_index("s") * 2 + lax.axis_index("c")
    pltpu.sync_copy(cs_hbm, cs_v)
    lanes = lax.broadcasted_iota(jnp.int32, (16,), 0)
    lanes_fs = lanes * _FS
    cs_vec = cs_v[...]
    s00, s01, s10, s11 = cs_vec[0], cs_vec[1], cs_vec[2], cs_vec[3]

    def pair_body(q, _):
        pair = wid * _PER_W + q
        b = pair // _C
        c = pair - b * _C
        s0 = jnp.where(b == 0, s00, s10)
        s1 = jnp.where(b == 0, s01, s11)

        def g_lo(jc):
            j0 = jc * _FC
            return jnp.minimum((jnp.maximum(s1 + j0 - 1, 0) // 8) * 8,
                               _G - _GR)

        def g_copy(jc, gb):
            return pltpu.make_async_copy(
                g_hbm.at[c, pl.ds(g_lo(jc), _GR), :], gb, sem)

        def f_chunk(jc, gb):
            j0 = jc * _FC
            lo = g_lo(jc)

            def f_rowgroup(jp, _):
                j = j0 + 4 * jp
                u = s1 + j
                bs = []
                for d in range(4):
                    bd = jnp.minimum(u + d, 256).astype(jnp.float32) * (1.0 / 256.0)
                    bv = jnp.full((16,), bd, jnp.float32)
                    bs.append((bv, 1.0 - bv))
                rs = [jnp.clip(u - 1 + d, 0, _G - 1) - lo for d in range(5)]
                base = j * _FS
                for v in range(16):
                    g = [gb[r, pl.ds(v * 16, 16)] for r in rs]
                    for d in range(4):
                        bv, bw = bs[d]
                        fbuf[pl.ds(base + d * _FS + v * 16, 16)] = (
                            bv * g[d] + bw * g[d + 1])
                return 0

            lax.fori_loop(0, _FC // 4, f_rowgroup, 0)

        g_copy(0, gbuf0).start()
        for jc in range(_G // _FC):
            gcur = gbuf0 if jc % 2 == 0 else gbuf1
            gnxt = gbuf1 if jc % 2 == 0 else gbuf0
            g_copy(jc, gcur).wait()
            if jc + 1 < _G // _FC:
                g_copy(jc + 1, gnxt).start()
            f_chunk(jc, gcur)
        gl = gbuf0 if (_G // _FC - 1) % 2 == 0 else gbuf1
        for v in range(16):
            fbuf[pl.ds(256 * _FS + v * 16, 16)] = gl[_GR - 1, pl.ds(v * 16, 16)]

        def e_compute(ec, bufs):
            bufa, bufb, bufc, bufd = bufs
            i0 = ec * _EC

            t0 = s0 + i0
            alpha0 = jnp.minimum(t0, 256).astype(jnp.float32) * (1.0 / 256.0)
            av0 = jnp.full((16,), alpha0, jnp.float32)
            aw0 = 1.0 - av0
            ia0 = lanes_fs + jnp.full((16,), jnp.clip(t0 - 1, 0, _G - 1), jnp.int32)
            ib0 = lanes_fs + jnp.full((16,), jnp.clip(t0, 0, _G - 1), jnp.int32)
            iay0 = ia0 + _FS
            iby0 = ib0 + _FS
            for v in range(16):
                fa = fbuf.at[pl.ds(v * 16 * _FS, _FPAD)]
                xa = plsc.load_gather(fa, [ia0])
                xb = plsc.load_gather(fa, [ib0])
                bufa[0, pl.ds(v * 16, 16)] = av0 * xa + aw0 * xb
                ya = plsc.load_gather(fa, [iay0])
                yb = plsc.load_gather(fa, [iby0])
                bufc[0, pl.ds(v * 16, 16)] = av0 * ya + aw0 * yb

            def group_taps(p):
                ii = 1 + 4 * p
                t = s0 + i0 + ii
                avs = []
                for d in range(4):
                    a = jnp.minimum(t + d, 256).astype(jnp.float32) * (1.0 / 256.0)
                    av = jnp.full((16,), a, jnp.float32)
                    avs.append((av, 1.0 - av))
                idx = [lanes_fs + jnp.full(
                           (16,), jnp.clip(t - 1 + d, 0, _G - 1), jnp.int32)
                       for d in range(5)]
                idy = [iv + _FS for iv in idx]
                return ii, avs, idx, idy

            def e_group(ii, avs, idx, idy, peel):
                for v in range(16):
                    fa = fbuf.at[pl.ds(v * 16 * _FS, _FPAD)]
                    x = [plsc.load_gather(fa, [iv]) for iv in idx]
                    for d in range(4):
                        av, aw = avs[d]
                        r = av * x[d] + aw * x[d + 1]
                        if not (peel and d == 3):
                            bufa[ii + d, pl.ds(v * 16, 16)] = r
                        bufb[ii + d - 1, pl.ds(v * 16, 16)] = r
                    y = [plsc.load_gather(fa, [iv]) for iv in idy]
                    for d in range(4):
                        av, aw = avs[d]
                        u = av * y[d] + aw * y[d + 1]
                        if not (peel and d == 3):
                            bufc[ii + d, pl.ds(v * 16, 16)] = u
                        bufd[ii + d - 1, pl.ds(v * 16, 16)] = u

            def e_rowgroup(p, _):
                ii, avs, idx, idy = group_taps(p)
                e_group(ii, avs, idx, idy, False)
                return 0

            lax.fori_loop(0, _EC // 4 - 1, e_rowgroup, 0)
            ii, avs, idx, idy = group_taps(_EC // 4 - 1)
            e_group(ii, avs, idx, idy, True)

        def e_copies(ec, bufs):
            i0 = ec * _EC
            for k, buf in enumerate(bufs):
                yield pltpu.make_async_copy(
                    buf.at[pl.ds(0, _EC), :],
                    out_hbm.at[b, k * _C + c, pl.ds(i0, _EC), :],
                    sem,
                )

        def e_start(ec, bufs):
            for cp in e_copies(ec, bufs):
                cp.start()

        def e_drain(ec, bufs):
            for cp in e_copies(ec, bufs):
                cp.wait()

        set0 = (bufa0, bufb0, bufc0, bufd0)
        set1 = (bufa1, bufb1, bufc1, bufd1)

        e_compute(0, set0)
        e_start(0, set0)
        e_compute(1, set1)
        e_start(1, set1)

        def e_ring(cp, _):
            ec = 2 * cp
            e_drain(ec - 2, set0)
            e_compute(ec, set0)
            e_start(ec, set0)
            e_drain(ec - 1, set1)
            e_compute(ec + 1, set1)
            e_start(ec + 1, set1)
            return 0

        lax.fori_loop(1, (_G // _EC) // 2, e_ring, 0)
        e_drain(_G // _EC - 2, set0)
        e_drain(_G // _EC - 1, set1)
        return 0

    lax.fori_loop(0, _PER_W, pair_body, 0)


@functools.partial(jax.jit, static_argnums=())
def _run(cs_pad, g2):
    mesh = plsc.VectorSubcoreMesh(core_axis_name="c", subcore_axis_name="s",
                                  num_cores=2, num_subcores=16)
    fn = pl.kernel(
        _body,
        out_type=jax.ShapeDtypeStruct((_B, 4 * _C, _G, _G), jnp.float32),
        mesh=mesh,
        scratch_types=[
            pltpu.VMEM((16,), jnp.int32),
            pltpu.VMEM((_GR, _G), jnp.float32),
            pltpu.VMEM((_GR, _G), jnp.float32),
            pltpu.VMEM((257 * _FS,), jnp.float32),
            pltpu.VMEM((_EC, _G), jnp.float32),
            pltpu.VMEM((_EC, _G), jnp.float32),
            pltpu.VMEM((_EC, _G), jnp.float32),
            pltpu.VMEM((_EC, _G), jnp.float32),
            pltpu.VMEM((_EC, _G), jnp.float32),
            pltpu.VMEM((_EC, _G), jnp.float32),
            pltpu.VMEM((_EC, _G), jnp.float32),
            pltpu.VMEM((_EC, _G), jnp.float32),
            pltpu.SemaphoreType.DMA,
        ],
        compiler_params=pltpu.CompilerParams(needs_layout_passes=False),
    )
    return fn(cs_pad, g2)


def kernel(coordinate_start, h, w, support_resolution_h, support_resolution_w, grid):
    del h, w, support_resolution_h, support_resolution_w
    cs_pad = jnp.zeros((16,), jnp.int32).at[0:4].set(coordinate_start.reshape(4))
    g2 = grid.reshape(_C, _G, _G)
    return _run(cs_pad, g2)
